# Initial kernel scaffold; baseline (speedup 1.0000x reference)
import jax
import jax.numpy as jnp
from jax.experimental import pallas as pl


def kernel(idx, table):
    raise NotImplementedError("write your pallas kernel here")


# speedup vs baseline: 1.0276x; 1.0276x over previous
import functools

import jax
import jax.numpy as jnp
from jax import lax
from jax.experimental import pallas as pl
from jax.experimental.pallas import tpu as pltpu
from jax.experimental.pallas import tpu_sc as plsc

_NC = 2
_NS = 16
_NW = _NC * _NS

_V = 1000
_D = 1000
_B = 1024
_S = 50
_N = _B * _S

_RPW = _N // _NW
_C = 40
_NCH = _RPW // _C


def _gather_body(idx_hbm, table_hbm, out_hbm, idx_v, buf0, buf1,
                 gsem, ssem0, ssem1):
    wid = lax.axis_index("s") * _NC + lax.axis_index("c")
    base = wid * _RPW

    pltpu.sync_copy(idx_hbm.at[wid], idx_v)

    bufs = (buf0, buf1)
    ssems = (ssem0, ssem1)

    def body(i, carry):
        for b in range(2):
            k = 2 * i + b

            @pl.when(i >= 1)
            def _wait_prev():
                pltpu.make_async_copy(
                    bufs[b],
                    out_hbm.at[pl.ds(base + (k - 2) * _C, _C)],
                    ssems[b],
                ).wait()

            pltpu.async_copy(
                table_hbm.at[idx_v.at[k]], bufs[b], gsem
            ).wait()

            pltpu.async_copy(
                bufs[b], out_hbm.at[pl.ds(base + k * _C, _C)], ssems[b]
            )
        return carry

    lax.fori_loop(0, _NCH // 2, body, None)

    pltpu.make_async_copy(
        buf0, out_hbm.at[pl.ds(base + (_RPW - 2 * _C), _C)], ssem0
    ).wait()
    pltpu.make_async_copy(
        buf1, out_hbm.at[pl.ds(base + (_RPW - _C), _C)], ssem1
    ).wait()


_mesh = plsc.VectorSubcoreMesh(
    core_axis_name="c", subcore_axis_name="s",
    num_cores=_NC, num_subcores=_NS,
)

_gather_call = functools.partial(
    pl.kernel,
    out_type=jax.ShapeDtypeStruct((_N, _D), jnp.float32),
    mesh=_mesh,
    compiler_params=pltpu.CompilerParams(use_tc_tiling_on_sc=False),
    scratch_types=[
        pltpu.VMEM((_NCH, _C), jnp.int32),
        pltpu.VMEM((_C, _D), jnp.float32),
        pltpu.VMEM((_C, _D), jnp.float32),
        pltpu.SemaphoreType.DMA,
        pltpu.SemaphoreType.DMA,
        pltpu.SemaphoreType.DMA,
    ],
)(_gather_body)


@jax.jit
def kernel(idx, table):
    idx3 = idx.reshape(_NW, _NCH, _C).astype(jnp.int32)
    out = _gather_call(idx3, table)
    return out.reshape(_B, _S, _V)
